# baseline (speedup 1.0000x reference)
import functools

import jax
import jax.numpy as jnp
from jax import lax
from jax.experimental import pallas as pl
from jax.experimental.pallas import tpu as pltpu
from jax.experimental.pallas import tpu_sc as plsc

_NC, _NS = 2, 16
_NW = _NC * _NS
_RS = 5120


def _sc_add(x2, w, rs):
    R, D = x2.shape
    L, _ = w.shape
    rows_per_w = rs // _NW
    CH = 16
    NCH = rows_per_w // CH

    mesh = plsc.VectorSubcoreMesh(core_axis_name="c", subcore_axis_name="s")

    @functools.partial(
        pl.kernel,
        mesh=mesh,
        out_type=jax.ShapeDtypeStruct((rs, D), jnp.float32),
        scratch_types=[
            pltpu.VMEM((CH, D), jnp.float32),
            pltpu.VMEM((CH, D), jnp.float32),
            pltpu.VMEM((CH, D), jnp.float32),
            pltpu.VMEM((CH, D), jnp.float32),
            pltpu.VMEM((CH, D), jnp.float32),
            pltpu.VMEM((CH, D), jnp.float32),
            pltpu.SemaphoreType.DMA,
            pltpu.SemaphoreType.DMA,
            pltpu.SemaphoreType.DMA,
            pltpu.SemaphoreType.DMA,
            pltpu.SemaphoreType.DMA,
            pltpu.SemaphoreType.DMA,
        ],
    )
    def run(x_hbm, w_hbm, o_hbm, xA, xB, wA, wB, oA, oB,
            sxA, sxB, swA, swB, soA, soB):
        wid = lax.axis_index("s") * _NC + lax.axis_index("c")
        base = wid * rows_per_w

        def in_copies(k, xbuf, wbuf, sx, sw):
            r0 = base + k * CH
            w0 = lax.rem(r0, L)
            return (
                pltpu.make_async_copy(x_hbm.at[pl.ds(r0, CH)], xbuf, sx),
                pltpu.make_async_copy(w_hbm.at[pl.ds(w0, CH)], wbuf, sw),
            )

        def out_copy(k, obuf, so):
            return pltpu.make_async_copy(
                obuf, o_hbm.at[pl.ds(base + k * CH, CH)], so
            )

        for cp in in_copies(0, xA, wA, sxA, swA):
            cp.start()
        for cp in in_copies(1, xB, wB, sxB, swB):
            cp.start()

        def step(k, xbuf, wbuf, obuf, sx, sw, so):
            for cp in in_copies(k, xbuf, wbuf, sx, sw):
                cp.wait()

            @pl.when(k >= 2)
            def _():
                out_copy(k - 2, obuf, so).wait()

            @pl.loop(0, CH)
            def _(r):
                rs_ = pl.ds(r, 1)
                for c in range(0, D, 16):
                    cs = pl.ds(c, 16)
                    obuf.at[rs_, cs][...] = (
                        xbuf.at[rs_, cs][...] + wbuf.at[rs_, cs][...]
                    )

            out_copy(k, obuf, so).start()

            @pl.when(k + 2 < NCH)
            def _():
                for cp in in_copies(k + 2, xbuf, wbuf, sx, sw):
                    cp.start()

        @pl.loop(0, NCH, step=2)
        def _(k):
            step(k, xA, wA, oA, sxA, swA, soA)
            step(k + 1, xB, wB, oB, sxB, swB, soB)

        out_copy(NCH - 2, oA, soA).wait()
        out_copy(NCH - 1, oB, soB).wait()

    return run(x2, w)


def _tc_body(x_ref, w_ref, o_ref, BL, L, row0):
    start = lax.rem(row0 + pl.program_id(0) * BL, L)
    o_ref[...] = x_ref[...] + w_ref[pl.ds(start, BL), :]


def _tc_add(x2, w, rs):
    R, D = x2.shape
    L, _ = w.shape
    BL = 1024
    nblk = (R - rs) // BL
    off = rs // BL
    return pl.pallas_call(
        functools.partial(_tc_body, BL=BL, L=L, row0=rs),
        grid=(nblk,),
        in_specs=[
            pl.BlockSpec((BL, D), lambda i: (i + off, 0)),
            pl.BlockSpec((L, D), lambda i: (0, 0)),
        ],
        out_specs=pl.BlockSpec((BL, D), lambda i: (i + off, 0)),
        out_shape=jax.ShapeDtypeStruct((R, D), jnp.float32),
        compiler_params=pltpu.CompilerParams(
            dimension_semantics=("arbitrary",),
        ),
    )(x2, w)


def kernel(x, weight):
    B, L, D = x.shape
    x2 = x.reshape(B * L, D)
    w = weight[:L]
    sc_part = _sc_add(x2, w, _RS)
    full = _tc_add(x2, w, _RS)
    out2 = lax.dynamic_update_slice(full, sc_part, (0, 0))
    return out2.reshape(B, L, D)
